# Initial kernel scaffold; baseline (speedup 1.0000x reference)
import jax
import jax.numpy as jnp
from jax.experimental import pallas as pl


def kernel(patch):
    raise NotImplementedError("write your pallas kernel here")


# speedup vs baseline: 5.2198x; 5.2198x over previous
import numpy as np
import jax
import jax.numpy as jnp
from jax import lax
from jax.experimental import pallas as pl
from jax.experimental.pallas import tpu as pltpu
from jax.experimental.pallas import tpu_sc as plsc

C = 16
N = 1024
ND = 32
D = 384
NAUG = 7

NC = 2
NS = 16
NW = NC * NS

ROWS = C * N
ROWS_PER_W = ROWS // NW
CHUNK = 128
NCH = ROWS_PER_W // CHUNK


def _build_tables():
    g = np.arange(N, dtype=np.int32).reshape(ND, ND)
    perms = [np.flip(g, 1).reshape(-1)]
    for k in range(1, 4):
        r = np.rot90(g, k)
        perms.append(r.reshape(-1))
        perms.append(np.flip(r, 1).reshape(-1))
    perm = np.stack(perms).astype(np.int32)
    inv = np.argsort(perm, axis=1).astype(np.int32)

    r = np.arange(ROWS, dtype=np.int64)
    c = r // N
    n = r % N
    dest = (np.arange(NAUG, dtype=np.int64)[:, None] * ROWS
            + c[None, :] * N + inv[:, n])
    dest = dest.transpose(1, 0).reshape(NW, NCH, CHUNK, NAUG)
    sidx = dest.transpose(0, 1, 3, 2).reshape(NW, NCH * NAUG, CHUNK)
    return np.asarray(sidx, dtype=np.int32), inv


_SIDX_NP, _INV_NP = _build_tables()


def _sc_body(patch_hbm, sidx_hbm, inv_hbm, out_hbm, invout_hbm,
             buf0, buf1, idx_v, inv_v, sem_in, sem_out):
    wid = lax.axis_index("c") * NS + lax.axis_index("s")
    base = wid * ROWS_PER_W

    pltpu.sync_copy(sidx_hbm.at[wid], idx_v)

    @pl.when(wid < NAUG)
    def _():
        pltpu.sync_copy(inv_hbm.at[wid], inv_v)
        pltpu.sync_copy(inv_v, invout_hbm.at[wid])

    bufs = (buf0, buf1)

    def load(j, buf):
        return pltpu.async_copy(
            patch_hbm.at[pl.ds(base + j * CHUNK, CHUNK)], buf, sem_in)

    scat = [None] * NCH
    ld = load(0, bufs[0])
    for j in range(NCH):
        ld.wait()
        buf = bufs[j % 2]
        if j + 1 < NCH:
            if j >= 1:
                for d in scat[j - 1]:
                    d.wait()
            ld = load(j + 1, bufs[(j + 1) % 2])
        scat[j] = [
            pltpu.async_copy(buf, out_hbm.at[idx_v.at[j * NAUG + a]], sem_out)
            for a in range(NAUG)
        ]
    for j in (NCH - 2, NCH - 1):
        for d in scat[j]:
            d.wait()


def kernel(patch):
    patch2d = patch.reshape(ROWS, D)
    sidx = jnp.asarray(_SIDX_NP)
    inv = jnp.asarray(_INV_NP)

    mesh = plsc.VectorSubcoreMesh(
        core_axis_name="c", subcore_axis_name="s",
        num_cores=NC, num_subcores=NS)

    out2d, invout = pl.kernel(
        _sc_body,
        out_type=(
            jax.ShapeDtypeStruct((NAUG * ROWS, D), jnp.float32),
            jax.ShapeDtypeStruct((NAUG, N), jnp.int32),
        ),
        mesh=mesh,
        scratch_types=[
            pltpu.VMEM((CHUNK, D), jnp.float32),
            pltpu.VMEM((CHUNK, D), jnp.float32),
            pltpu.VMEM((NCH * NAUG, CHUNK), jnp.int32),
            pltpu.VMEM((N,), jnp.int32),
            pltpu.SemaphoreType.DMA,
            pltpu.SemaphoreType.DMA,
        ],
    )(patch2d, sidx, inv)

    return out2d.reshape(NAUG, C, N, D), invout
